# baseline (speedup 1.0000x reference)
import functools

import jax
import jax.numpy as jnp
from jax import lax
from jax.experimental import pallas as pl
from jax.experimental.pallas import tpu as pltpu
from jax.experimental.pallas import tpu_sc as plsc

_B, _S, _D = 4, 4096, 768
_N = _B * _S
_NC, _NS = 2, 16
_NW = _NC * _NS
_SPW = _S // _NW
_CH = 32
_NSUB = _SPW // _CH
_NCHUNK = _NSUB * _B
_LANES = 16
_JV = _D // _LANES


def _make_emb_kernel():
    mesh = plsc.VectorSubcoreMesh(core_axis_name="c", subcore_axis_name="s")

    @functools.partial(
        pl.kernel,
        mesh=mesh,
        out_type=jax.ShapeDtypeStruct((_N, _D), jnp.float32),
        scratch_types=[
            pltpu.VMEM((_B, _SPW), jnp.int32),
            pltpu.VMEM((2, _CH, _D), jnp.float32),
            pltpu.VMEM((2, _CH, _D), jnp.float32),
            pltpu.SemaphoreType.DMA,
            pltpu.SemaphoreType.DMA,
            pltpu.SemaphoreType.DMA,
            pltpu.SemaphoreType.DMA,
            pltpu.SemaphoreType.DMA,
            pltpu.SemaphoreType.DMA,
            pltpu.SemaphoreType.DMA,
        ],
    )
    def emb(x_hbm, table_hbm, pe_hbm, out_hbm,
            idx_v, rows_v, pe_v,
            sem_i, sem_g0, sem_g1, sem_o0, sem_o1, sem_p0, sem_p1):
        wid = lax.axis_index("s") * _NC + lax.axis_index("c")
        s_base = wid * _SPW
        sems_g = (sem_g0, sem_g1)
        sems_o = (sem_o0, sem_o1)
        sems_p = (sem_p0, sem_p1)

        def coords(t):
            sub, b = t // _B, t % _B
            return sub, b, b * _S + s_base + sub * _CH

        def pe_src(sub):
            return pe_hbm.at[pl.ds(s_base + sub * _CH, _CH)]

        def gather_desc(t):
            sub, b, _ = coords(t)
            idx_sl = idx_v.at[b, pl.ds(sub * _CH, _CH)]
            return pltpu.make_async_copy(table_hbm.at[idx_sl],
                                         rows_v.at[t % 2], sems_g[t % 2])

        def out_desc(t):
            _, _, row0 = coords(t)
            return pltpu.make_async_copy(rows_v.at[t % 2],
                                         out_hbm.at[pl.ds(row0, _CH)],
                                         sems_o[t % 2])

        idx_desc = pltpu.make_async_copy(
            x_hbm.at[:, pl.ds(s_base, _SPW)], idx_v, sem_i)
        idx_desc.start()
        pltpu.async_copy(pe_src(0), pe_v.at[0], sem_p0)
        pltpu.async_copy(pe_src(1), pe_v.at[1], sem_p1)
        idx_desc.wait()

        def process(t):
            sub = t // _B
            p = sub % 2
            gather_desc(t).wait()

            def row_body(r, carry):
                for j in range(_JV):
                    sl = pl.ds(j * _LANES, _LANES)
                    plsc.addupdate(rows_v.at[t % 2, r, sl], pe_v[p, r, sl])
                return carry

            lax.fori_loop(0, _CH, row_body, 0)
            out_desc(t).start()

        for t in range(_NCHUNK):
            sub = t // _B
            p = sub % 2
            if t >= 2:
                out_desc(t - 2).wait()
            gather_desc(t).start()
            if t >= 1:
                process(t - 1)
            if t % _B == 0:
                pltpu.make_async_copy(pe_src(sub), pe_v.at[p],
                                      sems_p[p]).wait()
                if 1 <= sub and sub + 1 < _NSUB:
                    pltpu.async_copy(pe_src(sub + 1), pe_v.at[1 - p],
                                     sems_p[1 - p])

        process(_NCHUNK - 1)
        out_desc(_NCHUNK - 2).wait()
        out_desc(_NCHUNK - 1).wait()

    return emb


_emb = _make_emb_kernel()


def kernel(x, table, pe):
    out = _emb(x.astype(jnp.int32), table, pe)
    return out.reshape(_B, _S, _D)
